# Initial kernel scaffold; baseline (speedup 1.0000x reference)
import jax
import jax.numpy as jnp
from jax.experimental import pallas as pl


def kernel(x, weight):
    raise NotImplementedError("write your pallas kernel here")


# speedup vs baseline: 2.9628x; 2.9628x over previous
import functools

import jax
import jax.numpy as jnp
from jax import lax
from jax.experimental import pallas as pl
from jax.experimental.pallas import tpu as pltpu
from jax.experimental.pallas import tpu_sc as plsc

NC = 2
NS = 16
NW = NC * NS
C = 128


def _emb_call(n_chunks, D, dtype):
    mesh = plsc.VectorSubcoreMesh(
        core_axis_name="c", subcore_axis_name="s",
        num_cores=NC, num_subcores=NS,
    )

    @functools.partial(
        pl.kernel,
        mesh=mesh,
        out_type=jax.ShapeDtypeStruct((NW * n_chunks * C, D), dtype),
        scratch_types=[
            pltpu.VMEM((n_chunks, C), jnp.int32),
            pltpu.VMEM((C, D), dtype),
            pltpu.SemaphoreType.DMA,
        ],
    )
    def emb(idx_hbm, tbl_hbm, out_hbm, idx_v, rows_v, gsem):
        wid = lax.axis_index("s") * NC + lax.axis_index("c")
        cbase = wid * n_chunks
        pltpu.sync_copy(idx_hbm.at[wid], idx_v)

        def body(j, carry):
            pltpu.async_copy(tbl_hbm.at[idx_v.at[j]], rows_v, gsem).wait()
            pltpu.sync_copy(rows_v, out_hbm.at[pl.ds((cbase + j) * C, C)])
            return carry

        lax.fori_loop(0, n_chunks, body, 0)

    return emb


def kernel(x, weight):
    B0, B1 = x.shape
    V, D = weight.shape
    B = B0 * B1
    assert B % (NW * C) == 0
    n_chunks = B // (NW * C)
    idx = x.reshape(NW, n_chunks, C).astype(jnp.int32)
    out = _emb_call(n_chunks, D, weight.dtype)(idx, weight)
    return out.reshape(B0, B1, D)


# speedup vs baseline: 3.3395x; 1.1271x over previous
import functools

import jax
import jax.numpy as jnp
from jax import lax
from jax.experimental import pallas as pl
from jax.experimental.pallas import tpu as pltpu
from jax.experimental.pallas import tpu_sc as plsc

NC = 2
NS = 16
NW = NC * NS
C = 128
NBUF = 2


def _emb_call(n_chunks, D, dtype):
    mesh = plsc.VectorSubcoreMesh(
        core_axis_name="c", subcore_axis_name="s",
        num_cores=NC, num_subcores=NS,
    )

    @functools.partial(
        pl.kernel,
        mesh=mesh,
        out_type=jax.ShapeDtypeStruct((NW * n_chunks * C, D), dtype),
        scratch_types=[
            pltpu.VMEM((n_chunks, C), jnp.int32),
            pltpu.VMEM((NBUF, C, D), dtype),
            [pltpu.SemaphoreType.DMA] * NBUF,
        ],
    )
    def emb(idx_hbm, tbl_hbm, out_hbm, idx_v, rows_v, gsems):
        wid = lax.axis_index("s") * NC + lax.axis_index("c")
        cbase = wid * n_chunks
        pltpu.sync_copy(idx_hbm.at[wid], idx_v)
        bufs = [rows_v.at[b] for b in range(NBUF)]

        def start_gather(j, b):
            pltpu.async_copy(tbl_hbm.at[idx_v.at[j]], bufs[b], gsems[b])

        def chunk(j, b, start_next):
            pltpu.make_async_copy(
                tbl_hbm.at[idx_v.at[j]], bufs[b], gsems[b]).wait()
            pltpu.sync_copy(bufs[b], out_hbm.at[pl.ds((cbase + j) * C, C)])
            if start_next:
                start_gather(j + NBUF, b)

        for b in range(NBUF):
            start_gather(b, b)

        G = n_chunks // NBUF

        def outer(g, carry):
            for b in range(NBUF):
                chunk(g * NBUF + b, b, True)
            return carry

        lax.fori_loop(0, G - 1, outer, 0)
        for b in range(NBUF):
            chunk((G - 1) * NBUF + b, b, False)

    return emb


def kernel(x, weight):
    B0, B1 = x.shape
    V, D = weight.shape
    B = B0 * B1
    assert B % (NW * C) == 0
    n_chunks = B // (NW * C)
    idx = x.reshape(NW, n_chunks, C).astype(jnp.int32)
    out = _emb_call(n_chunks, D, weight.dtype)(idx, weight)
    return out.reshape(B0, B1, D)


# speedup vs baseline: 3.3449x; 1.0016x over previous
import functools

import jax
import jax.numpy as jnp
from jax import lax
from jax.experimental import pallas as pl
from jax.experimental.pallas import tpu as pltpu
from jax.experimental.pallas import tpu_sc as plsc

NC = 2
NS = 16
NW = NC * NS
C = 128
NBUF = 5


def _emb_call(n_chunks, D, dtype):
    mesh = plsc.VectorSubcoreMesh(
        core_axis_name="c", subcore_axis_name="s",
        num_cores=NC, num_subcores=NS,
    )

    @functools.partial(
        pl.kernel,
        mesh=mesh,
        out_type=jax.ShapeDtypeStruct((NW * n_chunks * C, D), dtype),
        scratch_types=[
            pltpu.VMEM((n_chunks, C), jnp.int32),
            pltpu.VMEM((NBUF, C, D), dtype),
            [pltpu.SemaphoreType.DMA] * NBUF,
        ],
    )
    def emb(idx_hbm, tbl_hbm, out_hbm, idx_v, rows_v, gsems):
        wid = lax.axis_index("s") * NC + lax.axis_index("c")
        cbase = wid * n_chunks
        pltpu.sync_copy(idx_hbm.at[wid], idx_v)
        bufs = [rows_v.at[b] for b in range(NBUF)]

        def start_gather(j, b):
            pltpu.async_copy(tbl_hbm.at[idx_v.at[j]], bufs[b], gsems[b])

        def chunk(j, b, start_next):
            pltpu.make_async_copy(
                tbl_hbm.at[idx_v.at[j]], bufs[b], gsems[b]).wait()
            pltpu.sync_copy(bufs[b], out_hbm.at[pl.ds((cbase + j) * C, C)])
            if start_next:
                start_gather(j + NBUF, b)

        for b in range(NBUF):
            start_gather(b, b)

        G = n_chunks // NBUF

        def outer(g, carry):
            for b in range(NBUF):
                chunk(g * NBUF + b, b, True)
            return carry

        lax.fori_loop(0, G - 1, outer, 0)
        for b in range(NBUF):
            chunk((G - 1) * NBUF + b, b, False)

    return emb


def kernel(x, weight):
    B0, B1 = x.shape
    V, D = weight.shape
    B = B0 * B1
    assert B % (NW * C) == 0
    n_chunks = B // (NW * C)
    idx = x.reshape(NW, n_chunks, C).astype(jnp.int32)
    out = _emb_call(n_chunks, D, weight.dtype)(idx, weight)
    return out.reshape(B0, B1, D)


# speedup vs baseline: 3.3492x; 1.0013x over previous
import functools

import jax
import jax.numpy as jnp
from jax import lax
from jax.experimental import pallas as pl
from jax.experimental.pallas import tpu as pltpu
from jax.experimental.pallas import tpu_sc as plsc

NC = 2
NS = 16
NW = NC * NS
C = 128
NBUF = 5
SLAG = 2


def _emb_call(n_chunks, D, dtype):
    mesh = plsc.VectorSubcoreMesh(
        core_axis_name="c", subcore_axis_name="s",
        num_cores=NC, num_subcores=NS,
    )

    @functools.partial(
        pl.kernel,
        mesh=mesh,
        out_type=jax.ShapeDtypeStruct((NW * n_chunks * C, D), dtype),
        scratch_types=[
            pltpu.VMEM((n_chunks, C), jnp.int32),
            pltpu.VMEM((NBUF, C, D), dtype),
            [pltpu.SemaphoreType.DMA] * NBUF,
            [pltpu.SemaphoreType.DMA] * NBUF,
        ],
    )
    def emb(idx_hbm, tbl_hbm, out_hbm, idx_v, rows_v, gsems, ssems):
        wid = lax.axis_index("s") * NC + lax.axis_index("c")
        cbase = wid * n_chunks
        pltpu.sync_copy(idx_hbm.at[wid], idx_v)
        bufs = [rows_v.at[b] for b in range(NBUF)]

        def start_gather(j, b):
            pltpu.async_copy(tbl_hbm.at[idx_v.at[j]], bufs[b], gsems[b])

        def wait_gather(j, b):
            pltpu.make_async_copy(
                tbl_hbm.at[idx_v.at[j]], bufs[b], gsems[b]).wait()

        def start_scatter(j, b):
            pltpu.async_copy(
                bufs[b], out_hbm.at[pl.ds((cbase + j) * C, C)], ssems[b])

        def wait_scatter(j, b):
            pltpu.make_async_copy(
                bufs[b], out_hbm.at[pl.ds((cbase + j) * C, C)],
                ssems[b]).wait()

        def chunk(j, b, do_prev):
            wait_gather(j, b)
            start_scatter(j, b)
            if do_prev:
                b2 = (b - SLAG) % NBUF
                j2 = j - SLAG
                wait_scatter(j2, b2)
                start_gather(j2 + NBUF, b2)

        n = n_chunks
        G = n // NBUF
        for m in range(NBUF):
            start_gather(m, m)
        for b in range(NBUF):
            chunk(b, b, b >= SLAG)

        def outer(g, carry):
            for b in range(NBUF):
                chunk(g * NBUF + b, b, True)
            return carry

        lax.fori_loop(1, G - 1, outer, 0)
        for b in range(NBUF):
            chunk((G - 1) * NBUF + b, b, b < SLAG)
        for t in range(NBUF):
            j2 = n - NBUF + t
            wait_scatter(j2, j2 % NBUF)

    return emb


def kernel(x, weight):
    B0, B1 = x.shape
    V, D = weight.shape
    B = B0 * B1
    assert B % (NW * C) == 0
    n_chunks = B // (NW * C)
    assert n_chunks % NBUF == 0 and n_chunks // NBUF >= 2
    idx = x.reshape(NW, n_chunks, C).astype(jnp.int32)
    out = _emb_call(n_chunks, D, weight.dtype)(idx, weight)
    return out.reshape(B0, B1, D)


# speedup vs baseline: 5.9923x; 1.7892x over previous
import functools

import jax
import jax.numpy as jnp
from jax import lax
from jax.experimental import pallas as pl
from jax.experimental.pallas import tpu as pltpu
from jax.experimental.pallas import tpu_sc as plsc

NC = 2
NS = 16
NW = NC * NS
NBUF = 8
SLAG = 3


def _emb_call(B0, B1, D, dtype):
    n = B0 // NW
    mesh = plsc.VectorSubcoreMesh(
        core_axis_name="c", subcore_axis_name="s",
        num_cores=NC, num_subcores=NS,
    )

    @functools.partial(
        pl.kernel,
        mesh=mesh,
        out_type=jax.ShapeDtypeStruct((B0, B1, D), dtype),
        scratch_types=[
            pltpu.VMEM((n, B1), jnp.int32),
            pltpu.VMEM((NBUF, B1, D), dtype),
            [pltpu.SemaphoreType.DMA] * NBUF,
            [pltpu.SemaphoreType.DMA] * NBUF,
        ],
    )
    def emb(idx_hbm, tbl_hbm, out_hbm, idx_v, rows_v, gsems, ssems):
        wid = lax.axis_index("s") * NC + lax.axis_index("c")
        rbase = wid * n
        pltpu.sync_copy(idx_hbm.at[pl.ds(rbase, n)], idx_v)
        bufs = [rows_v.at[b] for b in range(NBUF)]

        def start_gather(j, b):
            pltpu.async_copy(tbl_hbm.at[idx_v.at[j]], bufs[b], gsems[b])

        def wait_gather(j, b):
            pltpu.make_async_copy(
                tbl_hbm.at[idx_v.at[j]], bufs[b], gsems[b]).wait()

        def start_write(j, b):
            pltpu.async_copy(bufs[b], out_hbm.at[rbase + j], ssems[b])

        def wait_write(j, b):
            pltpu.make_async_copy(
                bufs[b], out_hbm.at[rbase + j], ssems[b]).wait()

        def chunk(j, b, do_prev):
            wait_gather(j, b)
            start_write(j, b)
            if do_prev:
                b2 = (b - SLAG) % NBUF
                j2 = j - SLAG
                wait_write(j2, b2)
                start_gather(j2 + NBUF, b2)

        G = n // NBUF
        for m in range(NBUF):
            start_gather(m, m)
        for b in range(NBUF):
            chunk(b, b, b >= SLAG)

        def outer(g, carry):
            for b in range(NBUF):
                chunk(g * NBUF + b, b, True)
            return carry

        lax.fori_loop(1, G - 1, outer, 0)
        for b in range(NBUF):
            chunk((G - 1) * NBUF + b, b, b < SLAG)
        for t in range(NBUF):
            j2 = n - NBUF + t
            wait_write(j2, j2 % NBUF)

    return emb


def kernel(x, weight):
    B0, B1 = x.shape
    V, D = weight.shape
    assert B0 % NW == 0 and (B0 // NW) % NBUF == 0 and B0 // (NW * NBUF) >= 2
    idx = x.astype(jnp.int32)
    return _emb_call(B0, B1, D, weight.dtype)(idx, weight)
